# baseline (speedup 1.0000x reference)
import jax
import jax.numpy as jnp
from jax.experimental import pallas as pl
from jax.experimental.pallas import tpu as pltpu

_CHUNK = 25
_N_CHUNKS = 400 // _CHUNK


def _body(winner_ref, mem_ref, val_ref, out_ref, buf_ref, sem_in, sem_out):
    n_chunks = _N_CHUNKS
    chunk = _CHUNK

    def fetch_row(k, b, r):
        row = k * chunk + r
        w = winner_ref[row]

        @pl.when(w >= 0)
        def _from_val():
            pltpu.make_async_copy(val_ref.at[w], buf_ref.at[b].at[r], sem_in.at[b]).start()

        @pl.when(w < 0)
        def _from_mem():
            pltpu.make_async_copy(mem_ref.at[row], buf_ref.at[b].at[r], sem_in.at[b]).start()

    for k in range(n_chunks):
        b = k % 2
        if k >= 2:
            pltpu.make_async_copy(
                buf_ref.at[b], out_ref.at[pl.ds((k - 2) * chunk, chunk)], sem_out.at[b]
            ).wait()
        jax.lax.fori_loop(0, chunk, lambda r, c, k=k, b=b: (fetch_row(k, b, r), c)[1], 0)
        jax.lax.fori_loop(
            0,
            chunk,
            lambda r, c, b=b: (
                pltpu.make_async_copy(mem_ref.at[0], buf_ref.at[b].at[0], sem_in.at[b]).wait(),
                c,
            )[1],
            0,
        )
        pltpu.make_async_copy(
            buf_ref.at[b], out_ref.at[pl.ds(k * chunk, chunk)], sem_out.at[b]
        ).start()

    for k in (n_chunks - 2, n_chunks - 1):
        b = k % 2
        pltpu.make_async_copy(
            buf_ref.at[b], out_ref.at[pl.ds(k * chunk, chunk)], sem_out.at[b]
        ).wait()


def kernel(mem, val, labels, slots):
    n_cls, length, n, c = mem.shape
    batch = val.shape[0]
    rows = n_cls * length

    ids = labels.astype(jnp.int32) * length + slots.astype(jnp.int32)
    matches = ids[None, :] == jnp.arange(rows, dtype=jnp.int32)[:, None]
    winner = jnp.max(
        jnp.where(matches, jnp.arange(batch, dtype=jnp.int32)[None, :], -1),
        axis=1,
    )

    out = pl.pallas_call(
        _body,
        grid_spec=pltpu.PrefetchScalarGridSpec(
            num_scalar_prefetch=1,
            grid=(1,),
            in_specs=[
                pl.BlockSpec(memory_space=pl.ANY),
                pl.BlockSpec(memory_space=pl.ANY),
            ],
            out_specs=pl.BlockSpec(memory_space=pl.ANY),
            scratch_shapes=[
                pltpu.VMEM((2, _CHUNK, 256, 128), jnp.float32),
                pltpu.SemaphoreType.DMA((2,)),
                pltpu.SemaphoreType.DMA((2,)),
            ],
        ),
        out_shape=jax.ShapeDtypeStruct((rows, n, c), mem.dtype),
        compiler_params=pltpu.CompilerParams(
            vmem_limit_bytes=100 * 1024 * 1024,
        ),
    )(winner, mem.reshape(rows, n, c), val)
    return out.reshape(mem.shape)


# speedup vs baseline: 1.4703x; 1.4703x over previous
import jax
import jax.numpy as jnp
from jax.experimental import pallas as pl
from jax.experimental.pallas import tpu as pltpu


_ROWS_PER_BLOCK = 100


def _body(winner_ref, mem_ref, val_ref, out_ref):
    g = pl.program_id(0)
    out_ref[...] = mem_ref[...]
    for r in range(_ROWS_PER_BLOCK):
        w = winner_ref[g * _ROWS_PER_BLOCK + r]

        @pl.when(w >= 0)
        def _use_val(w=w, r=r):
            out_ref[r] = val_ref[w]


def kernel(mem, val, labels, slots):
    n_cls, length, n, c = mem.shape
    batch = val.shape[0]
    rows = n_cls * length

    ids = labels.astype(jnp.int32) * length + slots.astype(jnp.int32)
    matches = ids[None, :] == jnp.arange(rows, dtype=jnp.int32)[:, None]
    winner = jnp.max(
        jnp.where(matches, jnp.arange(batch, dtype=jnp.int32)[None, :], -1),
        axis=1,
    )

    out = pl.pallas_call(
        _body,
        grid_spec=pltpu.PrefetchScalarGridSpec(
            num_scalar_prefetch=1,
            grid=(rows // _ROWS_PER_BLOCK,),
            in_specs=[
                pl.BlockSpec((_ROWS_PER_BLOCK, n, c), lambda g, w_ref: (g, 0, 0)),
                pl.BlockSpec((batch, n, c), lambda g, w_ref: (0, 0, 0)),
            ],
            out_specs=pl.BlockSpec((_ROWS_PER_BLOCK, n, c), lambda g, w_ref: (g, 0, 0)),
        ),
        out_shape=jax.ShapeDtypeStruct((rows, n, c), mem.dtype),
        compiler_params=pltpu.CompilerParams(
            vmem_limit_bytes=128 * 1024 * 1024,
        ),
    )(winner, mem.reshape(rows, n, c), val)
    return out.reshape(mem.shape)


# speedup vs baseline: 1.5335x; 1.0430x over previous
import jax
import jax.numpy as jnp
from jax.experimental import pallas as pl
from jax.experimental.pallas import tpu as pltpu

_ROWS_PER_BLOCK = 100


def _make_body(length):
    def _body(labels_ref, slots_ref, mem_ref, val_ref, out_ref):
        g = pl.program_id(0)
        out_ref[...] = mem_ref[...]
        batch = val_ref.shape[0]
        base = g * _ROWS_PER_BLOCK
        for b in range(batch):
            t = labels_ref[b] * length + slots_ref[b] - base

            @pl.when((t >= 0) & (t < _ROWS_PER_BLOCK))
            def _apply(t=t, b=b):
                out_ref[t] = val_ref[b]

    return _body


def kernel(mem, val, labels, slots):
    n_cls, length, n, c = mem.shape
    batch = val.shape[0]
    rows = n_cls * length

    out = pl.pallas_call(
        _make_body(length),
        grid_spec=pltpu.PrefetchScalarGridSpec(
            num_scalar_prefetch=2,
            grid=(rows // _ROWS_PER_BLOCK,),
            in_specs=[
                pl.BlockSpec((_ROWS_PER_BLOCK, n, c), lambda g, l_ref, s_ref: (g, 0, 0)),
                pl.BlockSpec((batch, n, c), lambda g, l_ref, s_ref: (0, 0, 0)),
            ],
            out_specs=pl.BlockSpec((_ROWS_PER_BLOCK, n, c), lambda g, l_ref, s_ref: (g, 0, 0)),
        ),
        out_shape=jax.ShapeDtypeStruct((rows, n, c), mem.dtype),
        compiler_params=pltpu.CompilerParams(
            vmem_limit_bytes=128 * 1024 * 1024,
        ),
    )(labels.astype(jnp.int32), slots.astype(jnp.int32), mem.reshape(rows, n, c), val)
    return out.reshape(mem.shape)
